# Initial kernel scaffold; baseline (speedup 1.0000x reference)
import jax
import jax.numpy as jnp
from jax.experimental import pallas as pl


def kernel(x, edge_index, batch, new_node_indices, focus_node_indices, Wc0, bc0, Wc1, bc1, Wc2, bc2, g0, beta0, g1, beta1, g2, beta2, Ws1, bs1, Ws2, bs2, Wa1, ba1, Wa2, ba2, Wb1, bb1, Wb2, bb2, We1, be1, We2, be2, We3, be3, Wt1, bt1, Wt2, bt2, Wt3, bt3):
    raise NotImplementedError("write your pallas kernel here")


# speedup vs baseline: 9.5834x; 9.5834x over previous
import functools
import jax
import jax.numpy as jnp
from jax import lax
from jax.experimental import pallas as pl
from jax.experimental.pallas import tpu as pltpu
from jax.experimental.pallas import tpu_sc as plsc

NN = 10000
EE = 320000
DD = 128
HH = 128
BBG = 256
NPAD = 10240
PB = 384
NC = 2
NS = 16
CH = 80
RB = 512

_f32 = jnp.float32


def _mesh():
    return plsc.VectorSubcoreMesh(core_axis_name="c", subcore_axis_name="s")


def _sc_hist(dst, batch_ext, z1d):
    epw = EE // (NC * NS)
    npw = NPAD // (NC * NS)
    rpt = NPAD // NS

    def body(dst_ref, batch_ref, z1d_ref, deg_out, cnt_out,
             idxv, onesv, degS, cntS):
        cid = lax.axis_index("c")
        sid = lax.axis_index("s")
        for k in range(CH // 16):
            onesv[pl.ds(16 * k, 16)] = jnp.ones((16,), _f32)
        pltpu.sync_copy(z1d_ref.at[pl.ds(sid * rpt, rpt)],
                        degS.at[pl.ds(sid * rpt, rpt)])

        @pl.when(sid == 0)
        def _():
            pltpu.sync_copy(z1d_ref.at[pl.ds(0, PB)], cntS)

        plsc.subcore_barrier()
        ebase = (cid * NS + sid) * epw

        def eloop(i, c):
            b = pl.multiple_of(ebase + i * CH, 8)
            pltpu.sync_copy(dst_ref.at[pl.ds(b, CH)], idxv)
            pltpu.sync_copy(onesv, degS.at[idxv], add=True)
            return c

        lax.fori_loop(0, epw // CH, eloop, 0)
        nbase = (cid * NS + sid) * npw

        def nloop(i, c):
            b = pl.multiple_of(nbase + i * CH, 8)
            pltpu.sync_copy(batch_ref.at[pl.ds(b, CH)], idxv)
            pltpu.sync_copy(onesv, cntS.at[idxv], add=True)
            return c

        lax.fori_loop(0, npw // CH, nloop, 0)
        plsc.subcore_barrier()
        pltpu.sync_copy(degS.at[pl.ds(sid * rpt, rpt)],
                        deg_out.at[cid, pl.ds(sid * rpt, rpt)])

        @pl.when(sid == 0)
        def _():
            pltpu.sync_copy(cntS, cnt_out.at[cid])

    return pl.kernel(
        body,
        out_type=(jax.ShapeDtypeStruct((NC, NPAD), _f32),
                  jax.ShapeDtypeStruct((NC, PB), _f32)),
        mesh=_mesh(),
        scratch_types=[
            pltpu.VMEM((CH,), jnp.int32),
            pltpu.VMEM((CH,), _f32),
            pltpu.VMEM_SHARED((NPAD,), _f32),
            pltpu.VMEM_SHARED((PB,), _f32),
        ],
    )(dst, batch_ext, z1d)


def _sc_edge_agg(zs, src, dst, z2d):
    epw = EE // (NC * NS)
    rpt = NPAD // NS

    def body(zs_ref, src_ref, dst_ref, z2d_ref, acc_out,
             sidx, didx, rows, accS):
        cid = lax.axis_index("c")
        sid = lax.axis_index("s")
        pltpu.sync_copy(z2d_ref.at[pl.ds(sid * rpt, rpt), :],
                        accS.at[pl.ds(sid * rpt, rpt), :])
        plsc.subcore_barrier()
        ebase = (cid * NS + sid) * epw

        def eloop(i, c):
            b = pl.multiple_of(ebase + i * CH, 8)
            pltpu.sync_copy(src_ref.at[pl.ds(b, CH)], sidx)
            pltpu.sync_copy(dst_ref.at[pl.ds(b, CH)], didx)
            pltpu.sync_copy(zs_ref.at[sidx], rows)
            pltpu.sync_copy(rows, accS.at[didx], add=True)
            return c

        lax.fori_loop(0, epw // CH, eloop, 0)
        plsc.subcore_barrier()
        pltpu.sync_copy(accS.at[pl.ds(sid * rpt, rpt), :],
                        acc_out.at[cid, pl.ds(sid * rpt, rpt), :])

    return pl.kernel(
        body,
        out_type=jax.ShapeDtypeStruct((NC, NPAD, HH), _f32),
        mesh=_mesh(),
        scratch_types=[
            pltpu.VMEM((CH,), jnp.int32),
            pltpu.VMEM((CH,), jnp.int32),
            pltpu.VMEM((CH, HH), _f32),
            pltpu.VMEM_SHARED((NPAD, HH), _f32),
        ],
    )(zs, src, dst, z2d)


def _sc_pool(node_emb, batch_ext, ntab, focus_idx, new_idx, z2d):
    npw = NPAD // (NC * NS)
    gpw = BBG // (NC * NS)
    ppt = PB // NS

    def body(ne_ref, batch_ref, ntab_ref, fidx_ref, nidx_ref, z2d_ref,
             pool_out, pair2_out, foc_out, new_out,
             idxv, idx2v, rows, rows2, sidx8, rows8, poolS):
        cid = lax.axis_index("c")
        sid = lax.axis_index("s")

        @pl.when(sid == 0)
        def _():
            pltpu.sync_copy(z2d_ref.at[pl.ds(0, PB), :], poolS)

        plsc.subcore_barrier()
        nbase = (cid * NS + sid) * npw

        def nloop(i, c):
            b = pl.multiple_of(nbase + i * CH, 8)
            pltpu.sync_copy(batch_ref.at[pl.ds(b, CH)], idxv)
            pltpu.sync_copy(ne_ref.at[pl.ds(b, CH), :], rows)
            pltpu.sync_copy(rows, poolS.at[idxv], add=True)
            pltpu.sync_copy(ntab_ref.at[idxv], idx2v)
            pltpu.sync_copy(ne_ref.at[idx2v], rows2)
            pltpu.sync_copy(rows2, pair2_out.at[pl.ds(b, CH), :])
            return c

        lax.fori_loop(0, npw // CH, nloop, 0)
        fb = (cid * NS + sid) * gpw
        pltpu.sync_copy(fidx_ref.at[pl.ds(fb, gpw)], sidx8)
        pltpu.sync_copy(ne_ref.at[sidx8], rows8)
        pltpu.sync_copy(rows8, foc_out.at[pl.ds(fb, gpw), :])
        pltpu.sync_copy(nidx_ref.at[pl.ds(fb, gpw)], sidx8)
        pltpu.sync_copy(ne_ref.at[sidx8], rows8)
        pltpu.sync_copy(rows8, new_out.at[pl.ds(fb, gpw), :])
        plsc.subcore_barrier()
        pltpu.sync_copy(poolS.at[pl.ds(sid * ppt, ppt), :],
                        pool_out.at[cid, pl.ds(sid * ppt, ppt), :])

    return pl.kernel(
        body,
        out_type=(jax.ShapeDtypeStruct((NC, PB, HH), _f32),
                  jax.ShapeDtypeStruct((NPAD, HH), _f32),
                  jax.ShapeDtypeStruct((BBG, HH), _f32),
                  jax.ShapeDtypeStruct((BBG, HH), _f32)),
        mesh=_mesh(),
        scratch_types=[
            pltpu.VMEM((CH,), jnp.int32),
            pltpu.VMEM((CH,), jnp.int32),
            pltpu.VMEM((CH, HH), _f32),
            pltpu.VMEM((CH, HH), _f32),
            pltpu.VMEM((gpw,), jnp.int32),
            pltpu.VMEM((gpw, HH), _f32),
            pltpu.VMEM_SHARED((PB, HH), _f32),
        ],
    )(node_emb, batch_ext, ntab, focus_idx, new_idx, z2d)


def _tc_prep(degp, x, W):
    def body(deg_ref, x_ref, W_ref, zs_ref, dinv_ref):
        deg = deg_ref[0, :] + deg_ref[1, :] + 1.0
        dinv = lax.rsqrt(jnp.maximum(deg, 1.0))
        z = jnp.dot(x_ref[...], W_ref[...], preferred_element_type=_f32)
        zs_ref[...] = z * dinv[:, None]
        dinv_ref[...] = dinv[:, None]

    return pl.pallas_call(
        body,
        grid=(NPAD // RB,),
        in_specs=[
            pl.BlockSpec((NC, RB), lambda i: (0, i)),
            pl.BlockSpec((RB, DD), lambda i: (i, 0)),
            pl.BlockSpec((DD, HH), lambda i: (0, 0)),
        ],
        out_specs=[
            pl.BlockSpec((RB, HH), lambda i: (i, 0)),
            pl.BlockSpec((RB, 1), lambda i: (i, 0)),
        ],
        out_shape=[
            jax.ShapeDtypeStruct((NPAD, HH), _f32),
            jax.ShapeDtypeStruct((NPAD, 1), _f32),
        ],
    )(degp, x, W)


def _tc_layer(accp, zs, dinv, h_in, b, g, beta, W_next):
    has_next = W_next is not None

    def body(accp_ref, zs_ref, dinv_ref, hin_ref, b_ref, g_ref, beta_ref,
             *rest):
        if has_next:
            Wn_ref, h_ref, zsn_ref = rest
        else:
            (h_ref,) = rest
        acc = accp_ref[0] + accp_ref[1] + zs_ref[...]
        dinv = dinv_ref[...]
        conv = acc * dinv + b_ref[...]
        m = jnp.mean(conv, axis=-1, keepdims=True)
        v = jnp.mean((conv - m) ** 2, axis=-1, keepdims=True)
        ln = (conv - m) / jnp.sqrt(v + 1e-5) * g_ref[...] + beta_ref[...]
        h = jnp.maximum(ln, 0.0) + hin_ref[...]
        h_ref[...] = h
        if has_next:
            zn = jnp.dot(h, Wn_ref[...], preferred_element_type=_f32)
            zsn_ref[...] = zn * dinv

    in_specs = [
        pl.BlockSpec((NC, RB, HH), lambda i: (0, i, 0)),
        pl.BlockSpec((RB, HH), lambda i: (i, 0)),
        pl.BlockSpec((RB, 1), lambda i: (i, 0)),
        pl.BlockSpec((RB, HH), lambda i: (i, 0)),
        pl.BlockSpec((1, HH), lambda i: (0, 0)),
        pl.BlockSpec((1, HH), lambda i: (0, 0)),
        pl.BlockSpec((1, HH), lambda i: (0, 0)),
    ]
    out_specs = [pl.BlockSpec((RB, HH), lambda i: (i, 0))]
    out_shape = [jax.ShapeDtypeStruct((NPAD, HH), _f32)]
    args = [accp, zs, dinv, h_in, b, g, beta]
    if has_next:
        in_specs.append(pl.BlockSpec((HH, HH), lambda i: (0, 0)))
        out_specs.append(pl.BlockSpec((RB, HH), lambda i: (i, 0)))
        out_shape.append(jax.ShapeDtypeStruct((NPAD, HH), _f32))
        args.append(W_next)

    res = pl.pallas_call(
        body,
        grid=(NPAD // RB,),
        in_specs=in_specs,
        out_specs=out_specs,
        out_shape=out_shape,
    )(*args)
    return res if has_next else (res[0], None)


def _tc_graph_heads(poolp, cntp, focus_emb, Ws1, bs1, Ws2, bs2,
                    Wa1, ba1, Wa2, ba2, Wb1, bb1, Wb2, bb2):
    def body(poolp_ref, cntp_ref, foc_ref, Ws1_ref, bs1_ref, Ws2_ref,
             bs2_ref, Wa1_ref, ba1_ref, Wa2_ref, ba2_ref, Wb1_ref, bb1_ref,
             Wb2_ref, bb2_ref, stop_ref, addn_ref, addb_ref):
        cnt = cntp_ref[0] + cntp_ref[1]
        pool = poolp_ref[0] + poolp_ref[1]
        ge = pool / jnp.maximum(cnt, 1.0)[:, None]

        def mlp2(z, W1r, b1r, W2r, b2r):
            z1 = jnp.maximum(
                jnp.dot(z, W1r[...], preferred_element_type=_f32)
                + b1r[...], 0.0)
            return jnp.dot(z1, W2r[...], preferred_element_type=_f32) \
                + b2r[...]

        stop_ref[...] = mlp2(ge, Ws1_ref, bs1_ref, Ws2_ref, bs2_ref)[:BBG]
        addn_ref[...] = mlp2(ge, Wa1_ref, ba1_ref, Wa2_ref, ba2_ref)[:BBG]
        addb_ref[...] = mlp2(foc_ref[...], Wb1_ref, bb1_ref, Wb2_ref,
                             bb2_ref)

    full = lambda s: pl.BlockSpec(s, lambda: tuple(0 for _ in s))
    return pl.pallas_call(
        body,
        in_specs=[
            full((NC, PB, HH)), full((NC, PB)), full((BBG, HH)),
            full((HH, HH)), full((1, HH)), full((HH, 1)), full((1, 1)),
            full((HH, HH)), full((1, HH)), full((HH, 10)), full((1, 10)),
            full((HH, HH)), full((1, HH)), full((HH, 4)), full((1, 4)),
        ],
        out_specs=[full((BBG, 1)), full((BBG, 10)), full((BBG, 4))],
        out_shape=[
            jax.ShapeDtypeStruct((BBG, 1), _f32),
            jax.ShapeDtypeStruct((BBG, 10), _f32),
            jax.ShapeDtypeStruct((BBG, 4), _f32),
        ],
    )(poolp, cntp, focus_emb, Ws1, bs1, Ws2, bs2,
      Wa1, ba1, Wa2, ba2, Wb1, bb1, Wb2, bb2)


def _tc_pair_heads(node_emb, pair2, We1a, We1b, be1, We2, be2, We3, be3,
                   Wt1a, Wt1b, bt1, Wt2, bt2, Wt3, bt3):
    def body(ne_ref, p2_ref, We1a_ref, We1b_ref, be1_ref, We2_ref, be2_ref,
             We3_ref, be3_ref, Wt1a_ref, Wt1b_ref, bt1_ref, Wt2_ref,
             bt2_ref, Wt3_ref, bt3_ref, oe_ref, ot_ref):
        ne = ne_ref[...]
        p2 = p2_ref[...]

        def head(W1a, W1b, b1, W2, b2, W3, b3):
            z = jnp.maximum(
                jnp.dot(ne, W1a[...], preferred_element_type=_f32)
                + jnp.dot(p2, W1b[...], preferred_element_type=_f32)
                + b1[...], 0.0)
            z = jnp.maximum(
                jnp.dot(z, W2[...], preferred_element_type=_f32)
                + b2[...], 0.0)
            return jnp.dot(z, W3[...], preferred_element_type=_f32) + b3[...]

        oe_ref[...] = head(We1a_ref, We1b_ref, be1_ref, We2_ref, be2_ref,
                           We3_ref, be3_ref)
        ot_ref[...] = head(Wt1a_ref, Wt1b_ref, bt1_ref, Wt2_ref, bt2_ref,
                           Wt3_ref, bt3_ref)

    hh2 = HH // 2
    return pl.pallas_call(
        body,
        grid=(NPAD // RB,),
        in_specs=[
            pl.BlockSpec((RB, HH), lambda i: (i, 0)),
            pl.BlockSpec((RB, HH), lambda i: (i, 0)),
            pl.BlockSpec((HH, HH), lambda i: (0, 0)),
            pl.BlockSpec((HH, HH), lambda i: (0, 0)),
            pl.BlockSpec((1, HH), lambda i: (0, 0)),
            pl.BlockSpec((HH, hh2), lambda i: (0, 0)),
            pl.BlockSpec((1, hh2), lambda i: (0, 0)),
            pl.BlockSpec((hh2, 1), lambda i: (0, 0)),
            pl.BlockSpec((1, 1), lambda i: (0, 0)),
            pl.BlockSpec((HH, HH), lambda i: (0, 0)),
            pl.BlockSpec((HH, HH), lambda i: (0, 0)),
            pl.BlockSpec((1, HH), lambda i: (0, 0)),
            pl.BlockSpec((HH, hh2), lambda i: (0, 0)),
            pl.BlockSpec((1, hh2), lambda i: (0, 0)),
            pl.BlockSpec((hh2, 4), lambda i: (0, 0)),
            pl.BlockSpec((1, 4), lambda i: (0, 0)),
        ],
        out_specs=[
            pl.BlockSpec((RB, 1), lambda i: (i, 0)),
            pl.BlockSpec((RB, 4), lambda i: (i, 0)),
        ],
        out_shape=[
            jax.ShapeDtypeStruct((NPAD, 1), _f32),
            jax.ShapeDtypeStruct((NPAD, 4), _f32),
        ],
    )(node_emb, pair2, We1a, We1b, be1, We2, be2, We3, be3,
      Wt1a, Wt1b, bt1, Wt2, bt2, Wt3, bt3)


def kernel(x, edge_index, batch, new_node_indices, focus_node_indices,
           Wc0, bc0, Wc1, bc1, Wc2, bc2,
           g0, beta0, g1, beta1, g2, beta2,
           Ws1, bs1, Ws2, bs2,
           Wa1, ba1, Wa2, ba2,
           Wb1, bb1, Wb2, bb2,
           We1, be1, We2, be2, We3, be3,
           Wt1, bt1, Wt2, bt2, Wt3, bt3):
    src = edge_index[0]
    dst = edge_index[1]
    xp = jnp.pad(x, ((0, NPAD - NN), (0, 0)))
    batch_ext = jnp.concatenate(
        [batch, jnp.full((NPAD - NN,), BBG, jnp.int32)])
    ntab = jnp.pad(new_node_indices, (0, PB - BBG))
    z1d = jnp.zeros((NPAD,), _f32)
    z2d = jnp.zeros((NPAD, HH), _f32)
    row = lambda v: v.reshape(1, -1)

    degp, cntp = _sc_hist(dst, batch_ext, z1d)
    zs, dinv = _tc_prep(degp, xp, Wc0)
    h = xp
    for (b, g, beta, Wn) in ((bc0, g0, beta0, Wc1),
                             (bc1, g1, beta1, Wc2),
                             (bc2, g2, beta2, None)):
        accp = _sc_edge_agg(zs, src, dst, z2d)
        h, zs = _tc_layer(accp, zs, dinv, h, row(b), row(g), row(beta), Wn)

    node_emb = h
    poolp, pair2, focus_emb, new_emb = _sc_pool(
        node_emb, batch_ext, ntab, focus_node_indices, new_node_indices,
        z2d)
    stop_logits, add_node_logits, add_bond_logits = _tc_graph_heads(
        poolp, cntp, focus_emb, Ws1, row(bs1), Ws2, row(bs2),
        Wa1, row(ba1), Wa2, row(ba2), Wb1, row(bb1), Wb2, row(bb2))
    edge_sel, bond_type = _tc_pair_heads(
        node_emb, pair2, We1[:HH], We1[HH:], row(be1), We2, row(be2),
        We3, row(be3), Wt1[:HH], Wt1[HH:], row(bt1), Wt2, row(bt2),
        Wt3, row(bt3))

    return (stop_logits, add_node_logits, add_bond_logits,
            node_emb[:NN], edge_sel[:NN], bond_type[:NN])
